# baseline (speedup 1.0000x reference)
import functools

import jax
import jax.numpy as jnp
from jax import lax
from jax.experimental import pallas as pl
from jax.experimental.pallas import tpu as pltpu
import jax.experimental.pallas.tpu_sc as plsc

N = 10000
D = 128
NPAD = 10240
NC = 2
NS = 16
NW = NC * NS
CHUNK = 32
ROWS_PER_TILE = NPAD // NS


def _zero_f32_ref(ref, nwords):
    z = jnp.zeros((16,), jnp.float32)

    def body(i, _):
        ref[pl.ds(i * 16, 16)] = z
        return 0

    lax.fori_loop(0, nwords // 16, body, 0)


def _make_deg_kernel(nch):
    mesh = plsc.VectorSubcoreMesh(core_axis_name="c", subcore_axis_name="s")

    @functools.partial(
        pl.kernel,
        out_type=jax.ShapeDtypeStruct((NC, NPAD), jnp.float32),
        mesh=mesh,
        scratch_types=[
            pltpu.VMEM((nch, CHUNK), jnp.int32),
            pltpu.VMEM((CHUNK,), jnp.float32),
            pltpu.VMEM((ROWS_PER_TILE,), jnp.float32),
            pltpu.VMEM_SHARED((NPAD,), jnp.float32),
        ],
    )
    def deg_kernel(col_hbm, deg_out, colidx_v, ones_v, zbuf_v, deg_sh):
        c = lax.axis_index("c")
        s = lax.axis_index("s")
        wid = c * NS + s

        one = jnp.ones((16,), jnp.float32)
        for i in range(CHUNK // 16):
            ones_v[pl.ds(i * 16, 16)] = one
        _zero_f32_ref(zbuf_v, ROWS_PER_TILE)
        pltpu.sync_copy(zbuf_v, deg_sh.at[pl.ds(s * ROWS_PER_TILE, ROWS_PER_TILE)])
        plsc.subcore_barrier()

        pltpu.sync_copy(col_hbm.at[wid], colidx_v)

        def body(j, _):
            pltpu.sync_copy(ones_v, deg_sh.at[colidx_v.at[j]], add=True)
            return 0

        lax.fori_loop(0, nch, body, 0)
        plsc.subcore_barrier()
        pltpu.sync_copy(
            deg_sh.at[pl.ds(s * ROWS_PER_TILE, ROWS_PER_TILE)],
            deg_out.at[c, pl.ds(s * ROWS_PER_TILE, ROWS_PER_TILE)],
        )

    return deg_kernel


def _make_agg_kernel(nch):
    mesh = plsc.VectorSubcoreMesh(core_axis_name="c", subcore_axis_name="s")

    @functools.partial(
        pl.kernel,
        out_type=jax.ShapeDtypeStruct((NC, NPAD, D), jnp.float32),
        mesh=mesh,
        scratch_types=[
            pltpu.VMEM((4, 2, CHUNK), jnp.int32),
            pltpu.VMEM((2, CHUNK, D), jnp.float32),
            pltpu.VMEM_SHARED((NPAD, D), jnp.float32),
            [pltpu.SemaphoreType.DMA] * 4,
            [pltpu.SemaphoreType.DMA] * 2,
            [pltpu.SemaphoreType.DMA] * 2,
        ],
    )
    def agg_kernel(ei_hbm, y_hbm, acc_out,
                   idx_v, rows_v, acc_sh, isems, rsems, wsems):
        c = lax.axis_index("c")
        s = lax.axis_index("s")
        wid = c * NS + s

        z = jnp.zeros((16,), jnp.float32)

        def zb(i, _):
            r = i // (D // 16)
            q = (i % (D // 16)) * 16
            rows_v[0, r, pl.ds(q, 16)] = z
            return 0

        lax.fori_loop(0, CHUNK * (D // 16), zb, 0)
        for zi in range(ROWS_PER_TILE // CHUNK):
            pltpu.sync_copy(
                rows_v.at[0],
                acc_sh.at[pl.ds(s * ROWS_PER_TILE + zi * CHUNK, CHUNK)],
            )
        plsc.subcore_barrier()

        def fire_idx(k, q):
            pltpu.async_copy(
                ei_hbm.at[wid, pl.ds(k, 1)], idx_v.at[pl.ds(q, 1)], isems[q]
            )

        def wait_idx(k, q):
            pltpu.make_async_copy(
                ei_hbm.at[wid, pl.ds(k, 1)], idx_v.at[pl.ds(q, 1)], isems[q]
            ).wait()

        def fire_gather(q, b):
            pltpu.async_copy(y_hbm.at[idx_v.at[q, 0]], rows_v.at[b], rsems[b])

        def wait_gather(q, b):
            pltpu.make_async_copy(
                y_hbm.at[idx_v.at[q, 0]], rows_v.at[b], rsems[b]
            ).wait()

        def fire_scatter(q, b):
            pltpu.async_copy(
                rows_v.at[b], acc_sh.at[idx_v.at[q, 1]], wsems[b], add=True
            )

        def wait_scatter(q, b):
            pltpu.make_async_copy(
                rows_v.at[b], acc_sh.at[idx_v.at[q, 1]], wsems[b]
            ).wait()

        fire_idx(0, 0)
        for k in (1, 2):
            if k < nch:
                fire_idx(k, k)
        wait_idx(0, 0)
        fire_gather(0, 0)

        def step(j, q):
            b = q % 2

            @pl.when(j + 1 < nch)
            def _():
                wait_idx(j + 1, (q + 1) % 4)

                @pl.when(j >= 1)
                def _():
                    wait_scatter((q + 3) % 4, 1 - b)

                fire_gather((q + 1) % 4, 1 - b)

            wait_gather(q, b)
            fire_scatter(q, b)

            @pl.when(j + 3 < nch)
            def _():
                fire_idx(j + 3, (q + 3) % 4)

        def body(j, _):
            for q in range(4):
                @pl.when(j % 4 == q)
                def _(q=q):
                    step(j, q)

            return 0

        lax.fori_loop(0, nch, body, 0)

        if nch >= 2:
            wait_scatter((nch - 2) % 4, (nch - 2) % 2)
        wait_scatter((nch - 1) % 4, (nch - 1) % 2)
        plsc.subcore_barrier()
        pltpu.sync_copy(
            acc_sh.at[pl.ds(s * ROWS_PER_TILE, ROWS_PER_TILE)],
            acc_out.at[c, pl.ds(s * ROWS_PER_TILE, ROWS_PER_TILE)],
        )

    return agg_kernel


def _y_body(x_ref, w_ref, deg_ref, y_ref):
    xl = lax.dot_general(
        x_ref[...], w_ref[...], (((1,), (1,)), ((), ())),
        preferred_element_type=jnp.float32,
    )
    deg = deg_ref[0, :] + deg_ref[1, :] + 1.0
    y_ref[...] = xl * lax.rsqrt(deg)[:, None]


def _tc_y(x_pad, W, deg2):
    blk = 2048
    grid = NPAD // blk
    return pl.pallas_call(
        _y_body,
        grid=(grid,),
        in_specs=[
            pl.BlockSpec((blk, D), lambda i: (i, 0)),
            pl.BlockSpec((D, D), lambda i: (0, 0)),
            pl.BlockSpec((NC, blk), lambda i: (0, i)),
        ],
        out_specs=pl.BlockSpec((blk, D), lambda i: (i, 0)),
        out_shape=jax.ShapeDtypeStruct((NPAD, D), jnp.float32),
    )(x_pad, W, deg2)


def _final_body(acc_ref, y_ref, deg_ref, b_ref, o_ref):
    deg = deg_ref[:, 0] + deg_ref[:, 1] + 1.0
    dinv = lax.rsqrt(deg)[:, None]
    o_ref[...] = dinv * (acc_ref[0] + acc_ref[1] + y_ref[...]) + b_ref[...]


def _tc_final(acc2, y, deg2t, b2):
    blk = 2000
    grid = N // blk
    return pl.pallas_call(
        _final_body,
        grid=(grid,),
        in_specs=[
            pl.BlockSpec((NC, blk, D), lambda i: (0, i, 0)),
            pl.BlockSpec((blk, D), lambda i: (i, 0)),
            pl.BlockSpec((blk, NC), lambda i: (i, 0)),
            pl.BlockSpec((1, D), lambda i: (0, 0)),
        ],
        out_specs=pl.BlockSpec((blk, D), lambda i: (i, 0)),
        out_shape=jax.ShapeDtypeStruct((N, D), jnp.float32),
    )(acc2, y, deg2t, b2)


@jax.jit
def kernel(x, edge_index, W, b):
    E = edge_index.shape[1]
    per_w = -(-E // (NW * CHUNK)) * CHUNK
    nch = per_w // CHUNK
    epad = per_w * NW

    pad = jnp.full((epad - E,), N, jnp.int32)
    row3 = jnp.concatenate([edge_index[0], pad]).reshape(NW, nch, 1, CHUNK)
    col3 = jnp.concatenate([edge_index[1], pad]).reshape(NW, nch, 1, CHUNK)
    ei3 = jnp.concatenate([row3, col3], axis=2)
    x_pad = jnp.pad(x, ((0, NPAD - N), (0, 0)))

    deg2 = _make_deg_kernel(nch)(col3.reshape(NW, nch, CHUNK))
    y = _tc_y(x_pad, W, deg2)
    acc2 = _make_agg_kernel(nch)(ei3, y)
    return _tc_final(acc2, y, deg2.T, b.reshape(1, D))


# speedup vs baseline: 1.1381x; 1.1381x over previous
import functools

import jax
import jax.numpy as jnp
from jax import lax
from jax.experimental import pallas as pl
from jax.experimental.pallas import tpu as pltpu
import jax.experimental.pallas.tpu_sc as plsc

N = 10000
D = 128
NPAD = 10240
NC = 2
NS = 16
NW = NC * NS
CHUNK = 64
ROWS_PER_TILE = NPAD // NS


def _zero_f32_ref(ref, nwords):
    z = jnp.zeros((16,), jnp.float32)

    def body(i, _):
        ref[pl.ds(i * 16, 16)] = z
        return 0

    lax.fori_loop(0, nwords // 16, body, 0)


def _make_deg_kernel(nch):
    mesh = plsc.VectorSubcoreMesh(core_axis_name="c", subcore_axis_name="s")

    @functools.partial(
        pl.kernel,
        out_type=jax.ShapeDtypeStruct((NC, NPAD), jnp.float32),
        mesh=mesh,
        scratch_types=[
            pltpu.VMEM((nch, CHUNK), jnp.int32),
            pltpu.VMEM((CHUNK,), jnp.float32),
            pltpu.VMEM((ROWS_PER_TILE,), jnp.float32),
            pltpu.VMEM_SHARED((NPAD,), jnp.float32),
        ],
    )
    def deg_kernel(col_hbm, deg_out, colidx_v, ones_v, zbuf_v, deg_sh):
        c = lax.axis_index("c")
        s = lax.axis_index("s")
        wid = c * NS + s

        one = jnp.ones((16,), jnp.float32)
        for i in range(CHUNK // 16):
            ones_v[pl.ds(i * 16, 16)] = one
        _zero_f32_ref(zbuf_v, ROWS_PER_TILE)
        pltpu.sync_copy(zbuf_v, deg_sh.at[pl.ds(s * ROWS_PER_TILE, ROWS_PER_TILE)])
        plsc.subcore_barrier()

        pltpu.sync_copy(col_hbm.at[wid], colidx_v)

        def body(j, _):
            pltpu.sync_copy(ones_v, deg_sh.at[colidx_v.at[j]], add=True)
            return 0

        lax.fori_loop(0, nch, body, 0)
        plsc.subcore_barrier()
        pltpu.sync_copy(
            deg_sh.at[pl.ds(s * ROWS_PER_TILE, ROWS_PER_TILE)],
            deg_out.at[c, pl.ds(s * ROWS_PER_TILE, ROWS_PER_TILE)],
        )

    return deg_kernel


def _make_agg_kernel(nch):
    mesh = plsc.VectorSubcoreMesh(core_axis_name="c", subcore_axis_name="s")

    @functools.partial(
        pl.kernel,
        out_type=jax.ShapeDtypeStruct((NC, NPAD, D), jnp.float32),
        mesh=mesh,
        scratch_types=[
            pltpu.VMEM((8, 2, CHUNK), jnp.int32),
            pltpu.VMEM((4, CHUNK, D), jnp.float32),
            pltpu.VMEM_SHARED((NPAD, D), jnp.float32),
            [pltpu.SemaphoreType.DMA] * 8,
            [pltpu.SemaphoreType.DMA] * 4,
            [pltpu.SemaphoreType.DMA] * 4,
        ],
    )
    def agg_kernel(ei_hbm, y_hbm, acc_out,
                   idx_v, rows_v, acc_sh, isems, rsems, wsems):
        c = lax.axis_index("c")
        s = lax.axis_index("s")
        wid = c * NS + s

        z = jnp.zeros((16,), jnp.float32)

        def zb(i, _):
            r = i // (D // 16)
            q = (i % (D // 16)) * 16
            rows_v[0, r, pl.ds(q, 16)] = z
            return 0

        lax.fori_loop(0, CHUNK * (D // 16), zb, 0)
        for zi in range(ROWS_PER_TILE // CHUNK):
            pltpu.sync_copy(
                rows_v.at[0],
                acc_sh.at[pl.ds(s * ROWS_PER_TILE + zi * CHUNK, CHUNK)],
            )
        plsc.subcore_barrier()

        def fire_idx(k, q):
            pltpu.async_copy(
                ei_hbm.at[wid, pl.ds(k, 1)], idx_v.at[pl.ds(q, 1)], isems[q]
            )

        def wait_idx(k, q):
            pltpu.make_async_copy(
                ei_hbm.at[wid, pl.ds(k, 1)], idx_v.at[pl.ds(q, 1)], isems[q]
            ).wait()

        def fire_gather(q, b):
            pltpu.async_copy(y_hbm.at[idx_v.at[q, 0]], rows_v.at[b], rsems[b])

        def wait_gather(q, b):
            pltpu.make_async_copy(
                y_hbm.at[idx_v.at[q, 0]], rows_v.at[b], rsems[b]
            ).wait()

        def fire_scatter(q, b):
            pltpu.async_copy(
                rows_v.at[b], acc_sh.at[idx_v.at[q, 1]], wsems[b], add=True
            )

        def wait_scatter(q, b):
            pltpu.make_async_copy(
                rows_v.at[b], acc_sh.at[idx_v.at[q, 1]], wsems[b]
            ).wait()

        for k in range(min(5, nch)):
            fire_idx(k, k)
        wait_idx(0, 0)
        fire_gather(0, 0)

        def step(j, m):
            b = m % 4

            @pl.when(j + 1 < nch)
            def _():
                wait_idx(j + 1, (m + 1) % 8)

                @pl.when(j >= 3)
                def _():
                    wait_scatter((m + 5) % 8, (b + 1) % 4)

                fire_gather((m + 1) % 8, (b + 1) % 4)

            wait_gather(m, b)
            fire_scatter(m, b)

            @pl.when(j + 5 < nch)
            def _():
                fire_idx(j + 5, (m + 5) % 8)

        def body(j, _):
            for m in range(8):
                @pl.when(j % 8 == m)
                def _(m=m):
                    step(j, m)

            return 0

        lax.fori_loop(0, nch, body, 0)

        for k in range(max(0, nch - 4), nch):
            wait_scatter(k % 8, k % 4)
        plsc.subcore_barrier()
        pltpu.sync_copy(
            acc_sh.at[pl.ds(s * ROWS_PER_TILE, ROWS_PER_TILE)],
            acc_out.at[c, pl.ds(s * ROWS_PER_TILE, ROWS_PER_TILE)],
        )

    return agg_kernel


def _y_body(x_ref, w_ref, deg_ref, y_ref):
    xl = lax.dot_general(
        x_ref[...], w_ref[...], (((1,), (1,)), ((), ())),
        preferred_element_type=jnp.float32,
    )
    deg = deg_ref[0, :] + deg_ref[1, :] + 1.0
    y_ref[...] = xl * lax.rsqrt(deg)[:, None]


def _tc_y(x_pad, W, deg2):
    blk = 2048
    grid = NPAD // blk
    return pl.pallas_call(
        _y_body,
        grid=(grid,),
        in_specs=[
            pl.BlockSpec((blk, D), lambda i: (i, 0)),
            pl.BlockSpec((D, D), lambda i: (0, 0)),
            pl.BlockSpec((NC, blk), lambda i: (0, i)),
        ],
        out_specs=pl.BlockSpec((blk, D), lambda i: (i, 0)),
        out_shape=jax.ShapeDtypeStruct((NPAD, D), jnp.float32),
    )(x_pad, W, deg2)


def _final_body(acc_ref, y_ref, deg_ref, b_ref, o_ref):
    deg = deg_ref[:, 0] + deg_ref[:, 1] + 1.0
    dinv = lax.rsqrt(deg)[:, None]
    o_ref[...] = dinv * (acc_ref[0] + acc_ref[1] + y_ref[...]) + b_ref[...]


def _tc_final(acc2, y, deg2t, b2):
    blk = 2000
    grid = N // blk
    return pl.pallas_call(
        _final_body,
        grid=(grid,),
        in_specs=[
            pl.BlockSpec((NC, blk, D), lambda i: (0, i, 0)),
            pl.BlockSpec((blk, D), lambda i: (i, 0)),
            pl.BlockSpec((blk, NC), lambda i: (i, 0)),
            pl.BlockSpec((1, D), lambda i: (0, 0)),
        ],
        out_specs=pl.BlockSpec((blk, D), lambda i: (i, 0)),
        out_shape=jax.ShapeDtypeStruct((N, D), jnp.float32),
    )(acc2, y, deg2t, b2)


@jax.jit
def kernel(x, edge_index, W, b):
    E = edge_index.shape[1]
    per_w = -(-E // (NW * CHUNK)) * CHUNK
    nch = per_w // CHUNK
    epad = per_w * NW

    pad = jnp.full((epad - E,), N, jnp.int32)
    row3 = jnp.concatenate([edge_index[0], pad]).reshape(NW, nch, 1, CHUNK)
    col3 = jnp.concatenate([edge_index[1], pad]).reshape(NW, nch, 1, CHUNK)
    ei3 = jnp.concatenate([row3, col3], axis=2)
    x_pad = jnp.pad(x, ((0, NPAD - N), (0, 0)))

    deg2 = _make_deg_kernel(nch)(col3.reshape(NW, nch, CHUNK))
    y = _tc_y(x_pad, W, deg2)
    acc2 = _make_agg_kernel(nch)(ei3, y)
    return _tc_final(acc2, y, deg2.T, b.reshape(1, D))


# speedup vs baseline: 1.1382x; 1.0001x over previous
import functools

import jax
import jax.numpy as jnp
from jax import lax
from jax.experimental import pallas as pl
from jax.experimental.pallas import tpu as pltpu
import jax.experimental.pallas.tpu_sc as plsc

N = 10000
D = 128
NPAD = 10240
NC = 2
NS = 16
NW = NC * NS
CHUNK = 64
ROWS_PER_TILE = NPAD // NS


def _zero_f32_ref(ref, nwords):
    z = jnp.zeros((16,), jnp.float32)

    def body(i, _):
        ref[pl.ds(i * 16, 16)] = z
        return 0

    lax.fori_loop(0, nwords // 16, body, 0)


def _make_deg_kernel(nch):
    mesh = plsc.VectorSubcoreMesh(core_axis_name="c", subcore_axis_name="s")

    @functools.partial(
        pl.kernel,
        out_type=jax.ShapeDtypeStruct((NC, NPAD), jnp.float32),
        mesh=mesh,
        scratch_types=[
            pltpu.VMEM((nch, CHUNK), jnp.int32),
            pltpu.VMEM((CHUNK,), jnp.float32),
            pltpu.VMEM((ROWS_PER_TILE,), jnp.float32),
            pltpu.VMEM_SHARED((NPAD,), jnp.float32),
            [pltpu.SemaphoreType.DMA] * 4,
        ],
    )
    def deg_kernel(col_hbm, deg_out, colidx_v, ones_v, zbuf_v, deg_sh, dsems):
        c = lax.axis_index("c")
        s = lax.axis_index("s")
        wid = c * NS + s

        one = jnp.ones((16,), jnp.float32)
        for i in range(CHUNK // 16):
            ones_v[pl.ds(i * 16, 16)] = one
        _zero_f32_ref(zbuf_v, ROWS_PER_TILE)
        pltpu.sync_copy(zbuf_v, deg_sh.at[pl.ds(s * ROWS_PER_TILE, ROWS_PER_TILE)])
        plsc.subcore_barrier()

        pltpu.sync_copy(col_hbm.at[wid], colidx_v)

        def body(j, _):
            for m in range(4):
                @pl.when(j % 4 == m)
                def _(m=m):
                    @pl.when(j >= 4)
                    def _():
                        pltpu.make_async_copy(
                            ones_v, deg_sh.at[colidx_v.at[j - 4]], dsems[m]
                        ).wait()

                    pltpu.async_copy(
                        ones_v, deg_sh.at[colidx_v.at[j]], dsems[m], add=True
                    )

            return 0

        lax.fori_loop(0, nch, body, 0)
        for k in range(max(0, nch - 4), nch):
            pltpu.make_async_copy(
                ones_v, deg_sh.at[colidx_v.at[k]], dsems[k % 4]
            ).wait()
        plsc.subcore_barrier()
        pltpu.sync_copy(
            deg_sh.at[pl.ds(s * ROWS_PER_TILE, ROWS_PER_TILE)],
            deg_out.at[c, pl.ds(s * ROWS_PER_TILE, ROWS_PER_TILE)],
        )

    return deg_kernel


def _make_agg_kernel(nch):
    mesh = plsc.VectorSubcoreMesh(core_axis_name="c", subcore_axis_name="s")

    @functools.partial(
        pl.kernel,
        out_type=jax.ShapeDtypeStruct((NC, NPAD, D), jnp.float32),
        mesh=mesh,
        scratch_types=[
            pltpu.VMEM((8, 2, CHUNK), jnp.int32),
            pltpu.VMEM((4, CHUNK, D), jnp.float32),
            pltpu.VMEM_SHARED((NPAD, D), jnp.float32),
            [pltpu.SemaphoreType.DMA] * 8,
            [pltpu.SemaphoreType.DMA] * 4,
            [pltpu.SemaphoreType.DMA] * 4,
        ],
    )
    def agg_kernel(ei_hbm, y_hbm, acc_out,
                   idx_v, rows_v, acc_sh, isems, rsems, wsems):
        c = lax.axis_index("c")
        s = lax.axis_index("s")
        wid = c * NS + s

        z = jnp.zeros((16,), jnp.float32)

        def zb(i, _):
            r = i // (D // 16)
            q = (i % (D // 16)) * 16
            rows_v[0, r, pl.ds(q, 16)] = z
            return 0

        lax.fori_loop(0, CHUNK * (D // 16), zb, 0)
        for zi in range(ROWS_PER_TILE // CHUNK):
            pltpu.sync_copy(
                rows_v.at[0],
                acc_sh.at[pl.ds(s * ROWS_PER_TILE + zi * CHUNK, CHUNK)],
            )
        plsc.subcore_barrier()

        def fire_idx(k, q):
            pltpu.async_copy(
                ei_hbm.at[wid, pl.ds(k, 1)], idx_v.at[pl.ds(q, 1)], isems[q]
            )

        def wait_idx(k, q):
            pltpu.make_async_copy(
                ei_hbm.at[wid, pl.ds(k, 1)], idx_v.at[pl.ds(q, 1)], isems[q]
            ).wait()

        def fire_gather(q, b):
            pltpu.async_copy(y_hbm.at[idx_v.at[q, 0]], rows_v.at[b], rsems[b])

        def wait_gather(q, b):
            pltpu.make_async_copy(
                y_hbm.at[idx_v.at[q, 0]], rows_v.at[b], rsems[b]
            ).wait()

        def fire_scatter(q, b):
            pltpu.async_copy(
                rows_v.at[b], acc_sh.at[idx_v.at[q, 1]], wsems[b], add=True
            )

        def wait_scatter(q, b):
            pltpu.make_async_copy(
                rows_v.at[b], acc_sh.at[idx_v.at[q, 1]], wsems[b]
            ).wait()

        for k in range(min(5, nch)):
            fire_idx(k, k)
        wait_idx(0, 0)
        fire_gather(0, 0)

        def step(j, m):
            b = m % 4

            @pl.when(j + 1 < nch)
            def _():
                wait_idx(j + 1, (m + 1) % 8)

                @pl.when(j >= 3)
                def _():
                    wait_scatter((m + 5) % 8, (b + 1) % 4)

                fire_gather((m + 1) % 8, (b + 1) % 4)

            wait_gather(m, b)
            fire_scatter(m, b)

            @pl.when(j + 5 < nch)
            def _():
                fire_idx(j + 5, (m + 5) % 8)

        def body(j, _):
            for m in range(8):
                @pl.when(j % 8 == m)
                def _(m=m):
                    step(j, m)

            return 0

        lax.fori_loop(0, nch, body, 0)

        for k in range(max(0, nch - 4), nch):
            wait_scatter(k % 8, k % 4)
        plsc.subcore_barrier()
        pltpu.sync_copy(
            acc_sh.at[pl.ds(s * ROWS_PER_TILE, ROWS_PER_TILE)],
            acc_out.at[c, pl.ds(s * ROWS_PER_TILE, ROWS_PER_TILE)],
        )

    return agg_kernel


def _y_body(x_ref, w_ref, deg_ref, y_ref):
    xl = lax.dot_general(
        x_ref[...], w_ref[...], (((1,), (1,)), ((), ())),
        preferred_element_type=jnp.float32,
    )
    deg = deg_ref[0, :] + deg_ref[1, :] + 1.0
    y_ref[...] = xl * lax.rsqrt(deg)[:, None]


def _tc_y(x_pad, W, deg2):
    blk = 2048
    grid = NPAD // blk
    return pl.pallas_call(
        _y_body,
        grid=(grid,),
        in_specs=[
            pl.BlockSpec((blk, D), lambda i: (i, 0)),
            pl.BlockSpec((D, D), lambda i: (0, 0)),
            pl.BlockSpec((NC, blk), lambda i: (0, i)),
        ],
        out_specs=pl.BlockSpec((blk, D), lambda i: (i, 0)),
        out_shape=jax.ShapeDtypeStruct((NPAD, D), jnp.float32),
    )(x_pad, W, deg2)


def _final_body(acc_ref, y_ref, deg_ref, b_ref, o_ref):
    deg = deg_ref[:, 0] + deg_ref[:, 1] + 1.0
    dinv = lax.rsqrt(deg)[:, None]
    o_ref[...] = dinv * (acc_ref[0] + acc_ref[1] + y_ref[...]) + b_ref[...]


def _tc_final(acc2, y, deg2t, b2):
    blk = 2000
    grid = N // blk
    return pl.pallas_call(
        _final_body,
        grid=(grid,),
        in_specs=[
            pl.BlockSpec((NC, blk, D), lambda i: (0, i, 0)),
            pl.BlockSpec((blk, D), lambda i: (i, 0)),
            pl.BlockSpec((blk, NC), lambda i: (i, 0)),
            pl.BlockSpec((1, D), lambda i: (0, 0)),
        ],
        out_specs=pl.BlockSpec((blk, D), lambda i: (i, 0)),
        out_shape=jax.ShapeDtypeStruct((N, D), jnp.float32),
    )(acc2, y, deg2t, b2)


@jax.jit
def kernel(x, edge_index, W, b):
    E = edge_index.shape[1]
    per_w = -(-E // (NW * CHUNK)) * CHUNK
    nch = per_w // CHUNK
    epad = per_w * NW

    pad = jnp.full((epad - E,), N, jnp.int32)
    row3 = jnp.concatenate([edge_index[0], pad]).reshape(NW, nch, 1, CHUNK)
    col3 = jnp.concatenate([edge_index[1], pad]).reshape(NW, nch, 1, CHUNK)
    ei3 = jnp.concatenate([row3, col3], axis=2)
    x_pad = jnp.pad(x, ((0, NPAD - N), (0, 0)))

    deg2 = _make_deg_kernel(nch)(col3.reshape(NW, nch, CHUNK))
    y = _tc_y(x_pad, W, deg2)
    acc2 = _make_agg_kernel(nch)(ei3, y)
    return _tc_final(acc2, y, deg2.T, b.reshape(1, D))
